# baseline (speedup 1.0000x reference)
import functools

import jax
import jax.numpy as jnp
from jax import lax
from jax.experimental import pallas as pl
from jax.experimental.pallas import tpu as pltpu
from jax.experimental.pallas import tpu_sc as plsc

_B = 4096
_F = 26
_V = 100000
_D = 16
_DD = 13
_H = 400
_EPS = 1e-3
_BF = _B * _F

_NC = 2
_NS = 16
_NW = _NC * _NS
_RPW = (_F * _D) // _NW
_LPW = _BF // _NW

_BB = 512
_SD = _F * _D
_DE = _DD * _D

_CT = (((0,), (0,)), ((), ()))


def _sc_rows_body(embT_hbm, idx_hbm, xT_out,
                  idx_v, row_v, out_v, sem, sem_out):
    wid = lax.axis_index("s") * _NC + lax.axis_index("c")

    def gather_row(src_row, p):
        pltpu.async_copy(src_row, row_v, sem).wait()

        def body(g, _):
            ig = idx_v[pl.ds(g * 16, 16)]
            out_v[p, pl.ds(g * 16, 16)] = plsc.load_gather(row_v, [ig])
            return 0

        lax.fori_loop(0, _B // 16, body, 0, unroll=8)

    flushes = [None, None]
    for j in range(_RPW):
        r = wid * _RPW + j
        p = j % 2
        if j == 0:
            pltpu.sync_copy(idx_hbm.at[pl.ds((r // _D) * _B, _B)], idx_v)
        else:
            @pl.when(r % _D == 0)
            def _():
                pltpu.sync_copy(idx_hbm.at[pl.ds((r // _D) * _B, _B)], idx_v)
        if flushes[p] is not None:
            flushes[p].wait()
        gather_row(embT_hbm.at[r], p)
        flushes[p] = pltpu.async_copy(out_v.at[p], xT_out.at[r], sem_out)

    flushes[_RPW % 2].wait()
    flushes[(_RPW + 1) % 2].wait()


def _sc_lin_body(lin2d_hbm, idx_hbm, linT_out, idx_v, row_v, out_v, sem):
    wid = lax.axis_index("s") * _NC + lax.axis_index("c")

    @pl.when(wid < _F)
    def _():
        pltpu.sync_copy(idx_hbm.at[pl.ds(wid * _B, _B)], idx_v)
        pltpu.async_copy(lin2d_hbm.at[wid], row_v, sem).wait()

        def body(g, _):
            ig = idx_v[pl.ds(g * 16, 16)]
            out_v[pl.ds(g * 16, 16)] = plsc.load_gather(row_v, [ig])
            return 0

        lax.fori_loop(0, _B // 16, body, 0, unroll=8)
        pltpu.sync_copy(out_v, linT_out.at[wid])


@functools.cache
def _sc_lin():
    return pl.kernel(
        _sc_lin_body,
        mesh=plsc.VectorSubcoreMesh(core_axis_name="c", subcore_axis_name="s"),
        out_type=jax.ShapeDtypeStruct((_F, _B), jnp.float32),
        scratch_types=[
            pltpu.VMEM((_B,), jnp.int32),
            pltpu.VMEM((_V,), jnp.float32),
            pltpu.VMEM((_B,), jnp.float32),
            pltpu.SemaphoreType.DMA,
        ],
        compiler_params=pltpu.CompilerParams(use_tc_tiling_on_sc=True,
                                             needs_layout_passes=False),
    )


@functools.cache
def _sc_rows():
    return pl.kernel(
        _sc_rows_body,
        mesh=plsc.VectorSubcoreMesh(core_axis_name="c", subcore_axis_name="s"),
        out_type=jax.ShapeDtypeStruct((_SD, _B), jnp.float32),
        scratch_types=[
            pltpu.VMEM((_B,), jnp.int32),
            pltpu.VMEM((_V,), jnp.float32),
            pltpu.VMEM((2, _B), jnp.float32),
            pltpu.SemaphoreType.DMA,
            pltpu.SemaphoreType.DMA,
        ],
        compiler_params=pltpu.CompilerParams(use_tc_tiling_on_sc=True,
                                             needs_layout_passes=False),
    )


def _field_sum_mat(n):
    r = lax.broadcasted_iota(jnp.int32, (n, _D), 0)
    c = lax.broadcasted_iota(jnp.int32, (n, _D), 1)
    return (r % _D == c).astype(jnp.float32)


def _tc_body(xT_ref, linT_ref, denseT_ref, wde_ref, bde_ref,
             w1s_ref, w1d_ref, s1_ref, t1_ref,
             w2_ref, s2_ref, t2_ref,
             w3_ref, s3_ref, t3_ref,
             wout_ref, wlin_ref, cst_ref, out_ref):
    f32 = jnp.float32
    xT = xT_ref[...]
    denseT = denseT_ref[...]
    de = lax.dot_general(denseT, wde_ref[...], _CT,
                         preferred_element_type=f32) + bde_ref[...]
    sum_emb = (lax.dot_general(xT, _field_sum_mat(_SD), _CT,
                               preferred_element_type=f32)
               + jnp.dot(de, _field_sum_mat(_DE), preferred_element_type=f32))
    sq_of_sum = jnp.sum(sum_emb * sum_emb, axis=1, keepdims=True)
    sum_of_sq = (lax.dot_general(xT * xT, jnp.ones((_SD, 1), f32), _CT,
                                 preferred_element_type=f32)
                 + jnp.sum(de * de, axis=1, keepdims=True))
    fm = 0.5 * (sq_of_sum - sum_of_sq)
    linear = (lax.dot_general(linT_ref[...], jnp.ones((_F, 1), f32), _CT,
                              preferred_element_type=f32)
              + lax.dot_general(denseT, wlin_ref[...], _CT,
                                preferred_element_type=f32))
    h = lax.dot_general(xT, w1s_ref[...], _CT, preferred_element_type=f32)
    h = h + jnp.dot(de, w1d_ref[...], preferred_element_type=f32)
    h = jnp.maximum(h * s1_ref[...] + t1_ref[...], 0.0)
    h = jnp.dot(h, w2_ref[...], preferred_element_type=f32)
    h = jnp.maximum(h * s2_ref[...] + t2_ref[...], 0.0)
    h = jnp.dot(h, w3_ref[...], preferred_element_type=f32)
    h = jnp.maximum(h * s3_ref[...] + t3_ref[...], 0.0)
    dnn = jnp.dot(h, wout_ref[...], preferred_element_type=f32)
    out_ref[...] = linear + fm + dnn + cst_ref[...]


def _tc_forward(xT, linT, denseT, wde, bde, w1s, w1d, s1, t1,
                w2, s2, t2, w3, s3, t3, wout, wlin, cst):
    nblk = _B // _BB
    full = lambda shape: pl.BlockSpec(shape, lambda i: (0, 0))
    col = lambda rows: pl.BlockSpec((rows, _BB), lambda i: (0, i))
    return pl.pallas_call(
        _tc_body,
        grid=(nblk,),
        in_specs=[
            col(_SD), col(_F), col(_DD),
            full((_DD, _DE)), full((1, _DE)),
            full((_SD, _H)), full((_DE, _H)), full((1, _H)), full((1, _H)),
            full((_H, _H)), full((1, _H)), full((1, _H)),
            full((_H, _H)), full((1, _H)), full((1, _H)),
            full((_H, 1)), full((_DD, 1)), full((1, 1)),
        ],
        out_specs=pl.BlockSpec((_BB, 1), lambda i: (i, 0)),
        out_shape=jax.ShapeDtypeStruct((_B, 1), jnp.float32),
    )(xT, linT, denseT, wde, bde, w1s, w1d, s1, t1,
      w2, s2, t2, w3, s3, t3, wout, wlin, cst)


def kernel(sparse_inputs, dense_inputs, emb_lin, W_lin, b_lin, linear_bias,
           emb, W_de, b_de, W1, b1, g1, be1, W2, b2, g2, be2,
           W3, b3, g3, be3, W_out, b_out):
    embT = emb.transpose(0, 2, 1).reshape(_SD, _V)
    sparse_T = sparse_inputs.T.astype(jnp.int32)
    idx_flat = sparse_T.reshape(-1)
    lin2d = emb_lin.reshape(_F, _V)

    xT = _sc_rows()(embT, idx_flat)
    linT = _sc_lin()(lin2d, idx_flat)
    denseT = dense_inputs.T

    inv = 1.0 / jnp.sqrt(1.0 + _EPS)
    s1 = (g1 * inv).reshape(1, _H)
    t1 = (be1 + b1 * g1 * inv).reshape(1, _H)
    s2 = (g2 * inv).reshape(1, _H)
    t2 = (be2 + b2 * g2 * inv).reshape(1, _H)
    s3 = (g3 * inv).reshape(1, _H)
    t3 = (be3 + b3 * g3 * inv).reshape(1, _H)
    wlin = jnp.sum(W_lin, axis=1).reshape(_DD, 1)
    cst = (jnp.sum(b_lin) + linear_bias[0] + b_out[0]).reshape(1, 1)
    return _tc_forward(
        xT, linT, denseT, W_de, b_de.reshape(1, _DE),
        W1[:_SD], W1[_SD:], s1, t1, W2, s2, t2, W3, s3, t3,
        W_out, wlin, cst)


# speedup vs baseline: 1.0460x; 1.0460x over previous
import functools

import jax
import jax.numpy as jnp
from jax import lax
from jax.experimental import pallas as pl
from jax.experimental.pallas import tpu as pltpu
from jax.experimental.pallas import tpu_sc as plsc

_B = 4096
_F = 26
_V = 100000
_D = 16
_DD = 13
_H = 400
_EPS = 1e-3
_BF = _B * _F

_NC = 2
_NS = 16
_NW = _NC * _NS
_RPW = (_F * _D) // _NW
_LPW = _BF // _NW

_BB = 1024
_SD = _F * _D
_DE = _DD * _D

_CT = (((0,), (0,)), ((), ()))


def _sc_rows_body(embT_hbm, lin2d_hbm, idx_hbm, xT_out, linT_out,
                  idx_v, row_v, out_v, sem, sem_out):
    wid = lax.axis_index("s") * _NC + lax.axis_index("c")

    def gather_row(src_row, p):
        pltpu.async_copy(src_row, row_v, sem).wait()

        def body(g, _):
            ig = idx_v[pl.ds(g * 16, 16)]
            out_v[p, pl.ds(g * 16, 16)] = plsc.load_gather(row_v, [ig])
            return 0

        lax.fori_loop(0, _B // 16, body, 0, unroll=8)

    flushes = [None, None]
    for j in range(_RPW):
        r = wid * _RPW + j
        p = j % 2
        if j == 0:
            pltpu.sync_copy(idx_hbm.at[pl.ds((r // _D) * _B, _B)], idx_v)
        else:
            @pl.when(r % _D == 0)
            def _():
                pltpu.sync_copy(idx_hbm.at[pl.ds((r // _D) * _B, _B)], idx_v)
        if flushes[p] is not None:
            flushes[p].wait()
        gather_row(embT_hbm.at[r], p)
        flushes[p] = pltpu.async_copy(out_v.at[p], xT_out.at[r], sem_out)

    flushes[_RPW % 2].wait()

    @pl.when(wid < _F)
    def _():
        pltpu.sync_copy(idx_hbm.at[pl.ds(wid * _B, _B)], idx_v)
        gather_row(lin2d_hbm.at[wid], _RPW % 2)
        pltpu.sync_copy(out_v.at[_RPW % 2], linT_out.at[wid])

    flushes[(_RPW + 1) % 2].wait()


@functools.cache
def _sc_rows():
    return pl.kernel(
        _sc_rows_body,
        mesh=plsc.VectorSubcoreMesh(core_axis_name="c", subcore_axis_name="s"),
        out_type=[
            jax.ShapeDtypeStruct((_SD, _B), jnp.float32),
            jax.ShapeDtypeStruct((_F, _B), jnp.float32),
        ],
        scratch_types=[
            pltpu.VMEM((_B,), jnp.int32),
            pltpu.VMEM((_V,), jnp.float32),
            pltpu.VMEM((2, _B), jnp.float32),
            pltpu.SemaphoreType.DMA,
            pltpu.SemaphoreType.DMA,
        ],
        compiler_params=pltpu.CompilerParams(use_tc_tiling_on_sc=True,
                                             needs_layout_passes=False),
    )


def _field_sum_mat(n):
    r = lax.broadcasted_iota(jnp.int32, (n, _D), 0)
    c = lax.broadcasted_iota(jnp.int32, (n, _D), 1)
    return (r % _D == c).astype(jnp.float32)


def _tc_body(xT_ref, linT_ref, denseT_ref, wde_ref, bde_ref,
             w1s_ref, w1d_ref, s1_ref, t1_ref,
             w2_ref, s2_ref, t2_ref,
             w3_ref, s3_ref, t3_ref,
             wout_ref, wlin_ref, cst_ref, out_ref):
    f32 = jnp.float32
    xT = xT_ref[...]
    denseT = denseT_ref[...]
    de = lax.dot_general(denseT, wde_ref[...], _CT,
                         preferred_element_type=f32) + bde_ref[...]
    sum_emb = (lax.dot_general(xT, _field_sum_mat(_SD), _CT,
                               preferred_element_type=f32)
               + jnp.dot(de, _field_sum_mat(_DE), preferred_element_type=f32))
    sq_of_sum = jnp.sum(sum_emb * sum_emb, axis=1, keepdims=True)
    sum_of_sq = (lax.dot_general(xT * xT, jnp.ones((_SD, 1), f32), _CT,
                                 preferred_element_type=f32)
                 + jnp.sum(de * de, axis=1, keepdims=True))
    fm = 0.5 * (sq_of_sum - sum_of_sq)
    linear = (lax.dot_general(linT_ref[...], jnp.ones((_F, 1), f32), _CT,
                              preferred_element_type=f32)
              + lax.dot_general(denseT, wlin_ref[...], _CT,
                                preferred_element_type=f32))
    h = lax.dot_general(xT, w1s_ref[...], _CT, preferred_element_type=f32)
    h = h + jnp.dot(de, w1d_ref[...], preferred_element_type=f32)
    h = jnp.maximum(h * s1_ref[...] + t1_ref[...], 0.0)
    h = jnp.dot(h, w2_ref[...], preferred_element_type=f32)
    h = jnp.maximum(h * s2_ref[...] + t2_ref[...], 0.0)
    h = jnp.dot(h, w3_ref[...], preferred_element_type=f32)
    h = jnp.maximum(h * s3_ref[...] + t3_ref[...], 0.0)
    dnn = jnp.dot(h, wout_ref[...], preferred_element_type=f32)
    out_ref[...] = linear + fm + dnn + cst_ref[...]


def _tc_forward(xT, linT, denseT, wde, bde, w1s, w1d, s1, t1,
                w2, s2, t2, w3, s3, t3, wout, wlin, cst):
    nblk = _B // _BB
    full = lambda shape: pl.BlockSpec(shape, lambda i: (0, 0))
    col = lambda rows: pl.BlockSpec((rows, _BB), lambda i: (0, i))
    return pl.pallas_call(
        _tc_body,
        grid=(nblk,),
        in_specs=[
            col(_SD), col(_F), col(_DD),
            full((_DD, _DE)), full((1, _DE)),
            full((_SD, _H)), full((_DE, _H)), full((1, _H)), full((1, _H)),
            full((_H, _H)), full((1, _H)), full((1, _H)),
            full((_H, _H)), full((1, _H)), full((1, _H)),
            full((_H, 1)), full((_DD, 1)), full((1, 1)),
        ],
        out_specs=pl.BlockSpec((_BB, 1), lambda i: (i, 0)),
        out_shape=jax.ShapeDtypeStruct((_B, 1), jnp.float32),
    )(xT, linT, denseT, wde, bde, w1s, w1d, s1, t1,
      w2, s2, t2, w3, s3, t3, wout, wlin, cst)


def kernel(sparse_inputs, dense_inputs, emb_lin, W_lin, b_lin, linear_bias,
           emb, W_de, b_de, W1, b1, g1, be1, W2, b2, g2, be2,
           W3, b3, g3, be3, W_out, b_out):
    embT = emb.transpose(0, 2, 1).reshape(_SD, _V)
    sparse_T = sparse_inputs.T.astype(jnp.int32)
    idx_flat = sparse_T.reshape(-1)
    lin2d = emb_lin.reshape(_F, _V)

    xT, linT = _sc_rows()(embT, lin2d, idx_flat)
    denseT = dense_inputs.T

    inv = 1.0 / jnp.sqrt(1.0 + _EPS)
    s1 = (g1 * inv).reshape(1, _H)
    t1 = (be1 + b1 * g1 * inv).reshape(1, _H)
    s2 = (g2 * inv).reshape(1, _H)
    t2 = (be2 + b2 * g2 * inv).reshape(1, _H)
    s3 = (g3 * inv).reshape(1, _H)
    t3 = (be3 + b3 * g3 * inv).reshape(1, _H)
    wlin = jnp.sum(W_lin, axis=1).reshape(_DD, 1)
    cst = (jnp.sum(b_lin) + linear_bias[0] + b_out[0]).reshape(1, 1)
    return _tc_forward(
        xT, linT, denseT, W_de, b_de.reshape(1, _DE),
        W1[:_SD], W1[_SD:], s1, t1, W2, s2, t2, W3, s3, t3,
        W_out, wlin, cst)


# speedup vs baseline: 1.0552x; 1.0088x over previous
import functools

import jax
import jax.numpy as jnp
from jax import lax
from jax.experimental import pallas as pl
from jax.experimental.pallas import tpu as pltpu
from jax.experimental.pallas import tpu_sc as plsc

_B = 4096
_F = 26
_V = 100000
_D = 16
_DD = 13
_H = 400
_EPS = 1e-3
_BF = _B * _F

_NC = 2
_NS = 16
_NW = _NC * _NS
_RPW = (_F * _D) // _NW
_LPW = _BF // _NW

_BB = 1024
_SD = _F * _D
_DE = _DD * _D

_CT = (((0,), (0,)), ((), ()))


def _sc_rows_body(embT_hbm, lin2d_hbm, idx_hbm, xT_out, linT_out,
                  idx_v, row_v, out_v, sem, sem_out):
    wid = lax.axis_index("s") * _NC + lax.axis_index("c")

    def gather(p):
        def body(g, _):
            ig = idx_v[pl.ds(g * 16, 16)]
            out_v[p, pl.ds(g * 16, 16)] = plsc.load_gather(row_v, [ig])
            return 0

        lax.fori_loop(0, _B // 16, body, 0, unroll=8)

    def gather_row(src_row, p):
        pltpu.async_copy(src_row, row_v, sem).wait()
        gather(p)

    flushes = [None, None]
    for j in range(_RPW):
        r = wid * _RPW + j
        p = j % 2
        cp = pltpu.async_copy(embT_hbm.at[r], row_v, sem)
        if j == 0:
            pltpu.sync_copy(idx_hbm.at[pl.ds((r // _D) * _B, _B)], idx_v)
        else:
            @pl.when(r % _D == 0)
            def _():
                pltpu.sync_copy(idx_hbm.at[pl.ds((r // _D) * _B, _B)], idx_v)
        if flushes[p] is not None:
            flushes[p].wait()
        cp.wait()
        gather(p)
        flushes[p] = pltpu.async_copy(out_v.at[p], xT_out.at[r], sem_out)

    flushes[_RPW % 2].wait()

    @pl.when(wid < _F)
    def _():
        pltpu.sync_copy(idx_hbm.at[pl.ds(wid * _B, _B)], idx_v)
        gather_row(lin2d_hbm.at[wid], _RPW % 2)
        pltpu.sync_copy(out_v.at[_RPW % 2], linT_out.at[wid])

    flushes[(_RPW + 1) % 2].wait()


@functools.cache
def _sc_rows():
    return pl.kernel(
        _sc_rows_body,
        mesh=plsc.VectorSubcoreMesh(core_axis_name="c", subcore_axis_name="s"),
        out_type=[
            jax.ShapeDtypeStruct((_SD, _B), jnp.float32),
            jax.ShapeDtypeStruct((_F, _B), jnp.float32),
        ],
        scratch_types=[
            pltpu.VMEM((_B,), jnp.int32),
            pltpu.VMEM((_V,), jnp.float32),
            pltpu.VMEM((2, _B), jnp.float32),
            pltpu.SemaphoreType.DMA,
            pltpu.SemaphoreType.DMA,
        ],
        compiler_params=pltpu.CompilerParams(use_tc_tiling_on_sc=True,
                                             needs_layout_passes=False),
    )


def _field_sum_mat(n):
    r = lax.broadcasted_iota(jnp.int32, (n, _D), 0)
    c = lax.broadcasted_iota(jnp.int32, (n, _D), 1)
    return (r % _D == c).astype(jnp.float32)


def _tc_body(xT_ref, linT_ref, denseT_ref, wde_ref, bde_ref,
             w1s_ref, w1d_ref, s1_ref, t1_ref,
             w2_ref, s2_ref, t2_ref,
             w3_ref, s3_ref, t3_ref,
             wout_ref, wlin_ref, cst_ref, out_ref):
    f32 = jnp.float32
    xT = xT_ref[...]
    denseT = denseT_ref[...]
    de = lax.dot_general(denseT, wde_ref[...], _CT,
                         preferred_element_type=f32) + bde_ref[...]
    sum_emb = (lax.dot_general(xT, _field_sum_mat(_SD), _CT,
                               preferred_element_type=f32)
               + jnp.dot(de, _field_sum_mat(_DE), preferred_element_type=f32))
    sq_of_sum = jnp.sum(sum_emb * sum_emb, axis=1, keepdims=True)
    sum_of_sq = (lax.dot_general(xT * xT, jnp.ones((_SD, 1), f32), _CT,
                                 preferred_element_type=f32)
                 + jnp.sum(de * de, axis=1, keepdims=True))
    fm = 0.5 * (sq_of_sum - sum_of_sq)
    linear = (lax.dot_general(linT_ref[...], jnp.ones((_F, 1), f32), _CT,
                              preferred_element_type=f32)
              + lax.dot_general(denseT, wlin_ref[...], _CT,
                                preferred_element_type=f32))
    h = lax.dot_general(xT, w1s_ref[...], _CT, preferred_element_type=f32)
    h = h + jnp.dot(de, w1d_ref[...], preferred_element_type=f32)
    h = jnp.maximum(h * s1_ref[...] + t1_ref[...], 0.0)
    h = jnp.dot(h, w2_ref[...], preferred_element_type=f32)
    h = jnp.maximum(h * s2_ref[...] + t2_ref[...], 0.0)
    h = jnp.dot(h, w3_ref[...], preferred_element_type=f32)
    h = jnp.maximum(h * s3_ref[...] + t3_ref[...], 0.0)
    dnn = jnp.dot(h, wout_ref[...], preferred_element_type=f32)
    out_ref[...] = linear + fm + dnn + cst_ref[...]


def _tc_forward(xT, linT, denseT, wde, bde, w1s, w1d, s1, t1,
                w2, s2, t2, w3, s3, t3, wout, wlin, cst):
    nblk = _B // _BB
    full = lambda shape: pl.BlockSpec(shape, lambda i: (0, 0))
    col = lambda rows: pl.BlockSpec((rows, _BB), lambda i: (0, i))
    return pl.pallas_call(
        _tc_body,
        grid=(nblk,),
        in_specs=[
            col(_SD), col(_F), col(_DD),
            full((_DD, _DE)), full((1, _DE)),
            full((_SD, _H)), full((_DE, _H)), full((1, _H)), full((1, _H)),
            full((_H, _H)), full((1, _H)), full((1, _H)),
            full((_H, _H)), full((1, _H)), full((1, _H)),
            full((_H, 1)), full((_DD, 1)), full((1, 1)),
        ],
        out_specs=pl.BlockSpec((_BB, 1), lambda i: (i, 0)),
        out_shape=jax.ShapeDtypeStruct((_B, 1), jnp.float32),
    )(xT, linT, denseT, wde, bde, w1s, w1d, s1, t1,
      w2, s2, t2, w3, s3, t3, wout, wlin, cst)


def kernel(sparse_inputs, dense_inputs, emb_lin, W_lin, b_lin, linear_bias,
           emb, W_de, b_de, W1, b1, g1, be1, W2, b2, g2, be2,
           W3, b3, g3, be3, W_out, b_out):
    embT = emb.transpose(0, 2, 1).reshape(_SD, _V)
    sparse_T = sparse_inputs.T.astype(jnp.int32)
    idx_flat = sparse_T.reshape(-1)
    lin2d = emb_lin.reshape(_F, _V)

    xT, linT = _sc_rows()(embT, lin2d, idx_flat)
    denseT = dense_inputs.T

    inv = 1.0 / jnp.sqrt(1.0 + _EPS)
    s1 = (g1 * inv).reshape(1, _H)
    t1 = (be1 + b1 * g1 * inv).reshape(1, _H)
    s2 = (g2 * inv).reshape(1, _H)
    t2 = (be2 + b2 * g2 * inv).reshape(1, _H)
    s3 = (g3 * inv).reshape(1, _H)
    t3 = (be3 + b3 * g3 * inv).reshape(1, _H)
    wlin = jnp.sum(W_lin, axis=1).reshape(_DD, 1)
    cst = (jnp.sum(b_lin) + linear_bias[0] + b_out[0]).reshape(1, 1)
    return _tc_forward(
        xT, linT, denseT, W_de, b_de.reshape(1, _DE),
        W1[:_SD], W1[_SD:], s1, t1, W2, s2, t2, W3, s3, t3,
        W_out, wlin, cst)
